# initial kernel scaffold (unmeasured)
import jax
import jax.numpy as jnp
from jax import lax
from jax.experimental import pallas as pl
from jax.experimental.pallas import tpu as pltpu

N_DEV = 4
SQ = 1024
SKV = 1024
H_TOT = 32
H_PER = 8
DH = 128
D_MODEL = 1024
SCALE = 0.08838834764831843
QBLK = 64


def _body(x_ref, wq_ref, k_ref, v_ref, wo_ref, out_ref,
          wq_ch, wo_ch, sq_send, sq_recv, so_send, so_recv):
    my = lax.axis_index("i")
    right = lax.rem(my + 1, N_DEV)
    left = lax.rem(my + N_DEV - 1, N_DEV)

    barrier = pltpu.get_barrier_semaphore()
    for nbr in (left, right):
        pl.semaphore_signal(barrier, inc=1, device_id=(nbr,),
                            device_id_type=pl.DeviceIdType.MESH)
    pl.semaphore_wait(barrier, 2)

    wq_ch[my] = wq_ref[...]
    wo_ch[my] = wo_ref[...]

    q_idx = my * SQ + lax.broadcasted_iota(jnp.int32, (SQ, SKV), 0)
    k_idx = lax.broadcasted_iota(jnp.int32, (SQ, SKV), 1)
    qb = q_idx // QBLK
    kb = k_idx // QBLK
    mask = (qb == kb) | (kb == 0) | (lax.rem(qb + kb, 3) == 0)
    bias = jnp.where(mask, 0.0, -1e9).astype(jnp.float32)

    x = x_ref[...]

    def compute_chunk(c, acc):
        def head_body(hl, acc):
            hg = c * H_PER + hl
            wq = wq_ch[c, :, pl.ds(hl * DH, DH)]
            q = lax.dot_general(x, wq, (((1,), (0,)), ((), ())),
                                preferred_element_type=jnp.float32)
            k = k_ref[:, pl.ds(hg * DH, DH)]
            s = lax.dot_general(q.astype(jnp.bfloat16), k,
                                (((1,), (1,)), ((), ())),
                                preferred_element_type=jnp.float32)
            s = s * SCALE + bias
            m = jnp.max(s, axis=-1, keepdims=True)
            w = jnp.exp(s - m)
            w = (w / jnp.sum(w, axis=-1, keepdims=True)).astype(jnp.bfloat16)
            v = v_ref[:, pl.ds(hg * DH, DH)]
            ctx = lax.dot_general(w, v, (((1,), (0,)), ((), ())),
                                  preferred_element_type=jnp.float32)
            wo = wo_ch[c, pl.ds(hl * DH, DH), :]
            acc = acc + lax.dot_general(ctx.astype(jnp.bfloat16), wo,
                                        (((1,), (0,)), ((), ())),
                                        preferred_element_type=jnp.float32)
            return acc
        return lax.fori_loop(0, H_PER, head_body, acc)

    acc = jnp.zeros((SQ, D_MODEL), jnp.float32)

    for h in range(N_DEV - 1):
        src = lax.rem(my - h + N_DEV, N_DEV)
        rq = pltpu.make_async_remote_copy(
            src_ref=wq_ch.at[src], dst_ref=wq_ch.at[src],
            send_sem=sq_send.at[h], recv_sem=sq_recv.at[h],
            device_id=(right,), device_id_type=pl.DeviceIdType.MESH)
        ro = pltpu.make_async_remote_copy(
            src_ref=wo_ch.at[src], dst_ref=wo_ch.at[src],
            send_sem=so_send.at[h], recv_sem=so_recv.at[h],
            device_id=(right,), device_id_type=pl.DeviceIdType.MESH)
        rq.start()
        ro.start()
        acc = compute_chunk(src, acc)
        rq.wait()
        ro.wait()

    acc = compute_chunk(lax.rem(my + 1, N_DEV), acc)
    out_ref[...] = acc


def kernel(x, Wq, K_ext, V_ext, Wo):
    xb = x[0].astype(jnp.bfloat16)
    wqb = Wq.astype(jnp.bfloat16)
    kb = K_ext[0].reshape(SKV, H_TOT * DH).astype(jnp.bfloat16)
    vb = V_ext[0].reshape(SKV, H_TOT * DH).astype(jnp.bfloat16)
    wob = Wo.astype(jnp.bfloat16)

    out = pl.pallas_call(
        _body,
        out_shape=jax.ShapeDtypeStruct((SQ, D_MODEL), jnp.float32),
        in_specs=[pl.BlockSpec(memory_space=pltpu.VMEM)] * 5,
        out_specs=pl.BlockSpec(memory_space=pltpu.VMEM),
        scratch_shapes=[
            pltpu.VMEM((N_DEV, D_MODEL, H_PER * DH), jnp.bfloat16),
            pltpu.VMEM((N_DEV, H_PER * DH, D_MODEL), jnp.bfloat16),
            pltpu.SemaphoreType.DMA((N_DEV - 1,)),
            pltpu.SemaphoreType.DMA((N_DEV - 1,)),
            pltpu.SemaphoreType.DMA((N_DEV - 1,)),
            pltpu.SemaphoreType.DMA((N_DEV - 1,)),
        ],
        compiler_params=pltpu.CompilerParams(collective_id=0),
    )(xb, wqb, kb, vb, wob)
    return out[None]


# baseline (device time: 243831 ns/iter reference)
import jax
import jax.numpy as jnp
from jax import lax
from jax.experimental import pallas as pl
from jax.experimental.pallas import tpu as pltpu

N_DEV = 4
SQ = 1024
SKV = 1024
H_TOT = 32
H_PER = 8
DH = 128
D_MODEL = 1024
SCALE = 0.08838834764831843
QBLK = 64


def _body(x_ref, wq_ref, k_ref, v_ref, wo_ref, out_ref,
          wq_ch, wo_ch, sq_send, sq_recv, so_send, so_recv):
    my = lax.axis_index("i")
    right = lax.rem(my + 1, N_DEV)
    left = lax.rem(my + N_DEV - 1, N_DEV)

    barrier = pltpu.get_barrier_semaphore()
    for nbr in (left, right):
        pl.semaphore_signal(barrier, inc=1, device_id=(nbr,),
                            device_id_type=pl.DeviceIdType.MESH)
    pl.semaphore_wait(barrier, 2)

    wq_ch[my] = wq_ref[...]
    wo_ch[my] = wo_ref[...]

    q_idx = my * SQ + lax.broadcasted_iota(jnp.int32, (SQ, SKV), 0)
    k_idx = lax.broadcasted_iota(jnp.int32, (SQ, SKV), 1)
    qb = q_idx // QBLK
    kb = k_idx // QBLK
    mask = (qb == kb) | (kb == 0) | (lax.rem(qb + kb, 3) == 0)
    bias = jnp.where(mask, 0.0, -1e9).astype(jnp.float32)

    x = x_ref[...]

    def compute_chunk(c, acc):
        def head_body(hl, acc):
            hg = c * H_PER + hl
            wq = wq_ch[c, :, pl.ds(hl * DH, DH)]
            q = lax.dot_general(x, wq, (((1,), (0,)), ((), ())),
                                preferred_element_type=jnp.float32)
            k = k_ref[:, pl.ds(hg * DH, DH)]
            s = lax.dot_general(q.astype(jnp.bfloat16), k,
                                (((1,), (1,)), ((), ())),
                                preferred_element_type=jnp.float32)
            s = s * SCALE + bias
            m = jnp.max(s, axis=-1, keepdims=True)
            w = jnp.exp(s - m)
            w = (w / jnp.sum(w, axis=-1, keepdims=True)).astype(jnp.bfloat16)
            v = v_ref[:, pl.ds(hg * DH, DH)]
            ctx = lax.dot_general(w, v, (((1,), (0,)), ((), ())),
                                  preferred_element_type=jnp.float32)
            wo = wo_ch[c, pl.ds(hl * DH, DH), :]
            acc = acc + lax.dot_general(ctx.astype(jnp.bfloat16), wo,
                                        (((1,), (0,)), ((), ())),
                                        preferred_element_type=jnp.float32)
            return acc
        return lax.fori_loop(0, H_PER, head_body, acc)

    acc = jnp.zeros((SQ, D_MODEL), jnp.float32)

    for h in range(N_DEV - 1):
        src = lax.rem(my - h + N_DEV, N_DEV)
        rq = pltpu.make_async_remote_copy(
            src_ref=wq_ch.at[src], dst_ref=wq_ch.at[src],
            send_sem=sq_send.at[h], recv_sem=sq_recv.at[h],
            device_id=(right,), device_id_type=pl.DeviceIdType.MESH)
        ro = pltpu.make_async_remote_copy(
            src_ref=wo_ch.at[src], dst_ref=wo_ch.at[src],
            send_sem=so_send.at[h], recv_sem=so_recv.at[h],
            device_id=(right,), device_id_type=pl.DeviceIdType.MESH)
        rq.start()
        ro.start()
        acc = compute_chunk(src, acc)
        rq.wait()
        ro.wait()

    acc = compute_chunk(lax.rem(my + 1, N_DEV), acc)
    out_ref[...] = acc


def kernel(x, Wq, K_ext, V_ext, Wo):
    xb = x[0].astype(jnp.bfloat16)
    wqb = Wq.astype(jnp.bfloat16)
    kb = K_ext[0].reshape(SKV, H_TOT * DH).astype(jnp.bfloat16)
    vb = V_ext[0].reshape(SKV, H_TOT * DH).astype(jnp.bfloat16)
    wob = Wo.astype(jnp.bfloat16)

    out = pl.pallas_call(
        _body,
        out_shape=jax.ShapeDtypeStruct((SQ, D_MODEL), jnp.float32),
        in_specs=[pl.BlockSpec(memory_space=pltpu.VMEM)] * 5,
        out_specs=pl.BlockSpec(memory_space=pltpu.VMEM),
        scratch_shapes=[
            pltpu.VMEM((N_DEV, D_MODEL, H_PER * DH), jnp.bfloat16),
            pltpu.VMEM((N_DEV, H_PER * DH, D_MODEL), jnp.bfloat16),
            pltpu.SemaphoreType.DMA((N_DEV - 1,)),
            pltpu.SemaphoreType.DMA((N_DEV - 1,)),
            pltpu.SemaphoreType.DMA((N_DEV - 1,)),
            pltpu.SemaphoreType.DMA((N_DEV - 1,)),
        ],
        compiler_params=pltpu.CompilerParams(
            collective_id=0, vmem_limit_bytes=100 * 1024 * 1024),
    )(xb, wqb, kb, vb, wob)
    return out[None]


# device time: 213881 ns/iter; 1.1400x vs baseline; 1.1400x over previous
import jax
import jax.numpy as jnp
from jax import lax
from jax.experimental import pallas as pl
from jax.experimental.pallas import tpu as pltpu

N_DEV = 4
SQ = 1024
SKV = 1024
H_TOT = 32
H_PER = 8
DH = 128
D_MODEL = 1024
SCALE = 0.08838834764831843
QBLK = 64


def _body(x_ref, wq_ref, k_ref, v_ref, wo_ref, out_ref,
          wq_ch, wo_ch, ctx_buf, bias_ref,
          sq_send, sq_recv, so_send, so_recv):
    my = lax.axis_index("i")
    right = lax.rem(my + 1, N_DEV)
    left = lax.rem(my + N_DEV - 1, N_DEV)

    barrier = pltpu.get_barrier_semaphore()
    for nbr in (left, right):
        pl.semaphore_signal(barrier, inc=1, device_id=(nbr,),
                            device_id_type=pl.DeviceIdType.MESH)
    pl.semaphore_wait(barrier, 2)

    wq_ch[my] = wq_ref[...].astype(jnp.bfloat16)
    wo_ch[my] = wo_ref[...].astype(jnp.bfloat16)

    q_idx = my * SQ + lax.broadcasted_iota(jnp.int32, (SQ, SKV), 0)
    k_idx = lax.broadcasted_iota(jnp.int32, (SQ, SKV), 1)
    qb = q_idx // QBLK
    kb = k_idx // QBLK
    mask = (qb == kb) | (kb == 0) | (lax.rem(qb + kb, 3) == 0)
    bias_ref[...] = jnp.where(mask, 0.0, -1e9).astype(jnp.float32)

    out_ref[...] = jnp.zeros((SQ, D_MODEL), jnp.float32)
    xb = x_ref[...].astype(jnp.bfloat16)

    def compute_chunk(c):
        def head_body(hl, _):
            hg = c * H_PER + hl
            wq = wq_ch[c, :, pl.ds(hl * DH, DH)]
            q = lax.dot_general(xb, wq, (((1,), (0,)), ((), ())),
                                preferred_element_type=jnp.float32)
            q = (q * SCALE).astype(jnp.bfloat16)
            k = k_ref[:, pl.ds(hg * DH, DH)]
            s = lax.dot_general(q, k, (((1,), (1,)), ((), ())),
                                preferred_element_type=jnp.float32)
            w = jnp.exp(s + bias_ref[...])
            r = 1.0 / jnp.sum(w, axis=-1, keepdims=True)
            v = v_ref[:, pl.ds(hg * DH, DH)]
            ctx = lax.dot_general(w.astype(jnp.bfloat16), v,
                                  (((1,), (0,)), ((), ())),
                                  preferred_element_type=jnp.float32)
            ctx_buf[:, pl.ds(hl * DH, DH)] = (ctx * r).astype(jnp.bfloat16)
            return 0

        lax.fori_loop(0, H_PER, head_body, 0)
        out_ref[...] += lax.dot_general(ctx_buf[...], wo_ch[c],
                                        (((1,), (0,)), ((), ())),
                                        preferred_element_type=jnp.float32)

    for h in range(N_DEV - 1):
        src = lax.rem(my - h + N_DEV, N_DEV)
        rq = pltpu.make_async_remote_copy(
            src_ref=wq_ch.at[src], dst_ref=wq_ch.at[src],
            send_sem=sq_send.at[h], recv_sem=sq_recv.at[h],
            device_id=(right,), device_id_type=pl.DeviceIdType.MESH)
        ro = pltpu.make_async_remote_copy(
            src_ref=wo_ch.at[src], dst_ref=wo_ch.at[src],
            send_sem=so_send.at[h], recv_sem=so_recv.at[h],
            device_id=(right,), device_id_type=pl.DeviceIdType.MESH)
        rq.start()
        ro.start()
        compute_chunk(src)
        rq.wait()
        ro.wait()

    compute_chunk(lax.rem(my + 1, N_DEV))


def kernel(x, Wq, K_ext, V_ext, Wo):
    kb = K_ext[0].reshape(SKV, H_TOT * DH).astype(jnp.bfloat16)
    vb = V_ext[0].reshape(SKV, H_TOT * DH).astype(jnp.bfloat16)

    out = pl.pallas_call(
        _body,
        out_shape=jax.ShapeDtypeStruct((SQ, D_MODEL), jnp.float32),
        in_specs=[pl.BlockSpec(memory_space=pltpu.VMEM)] * 5,
        out_specs=pl.BlockSpec(memory_space=pltpu.VMEM),
        scratch_shapes=[
            pltpu.VMEM((N_DEV, D_MODEL, H_PER * DH), jnp.bfloat16),
            pltpu.VMEM((N_DEV, H_PER * DH, D_MODEL), jnp.bfloat16),
            pltpu.VMEM((SQ, H_PER * DH), jnp.bfloat16),
            pltpu.VMEM((SQ, SKV), jnp.float32),
            pltpu.SemaphoreType.DMA((N_DEV - 1,)),
            pltpu.SemaphoreType.DMA((N_DEV - 1,)),
            pltpu.SemaphoreType.DMA((N_DEV - 1,)),
            pltpu.SemaphoreType.DMA((N_DEV - 1,)),
        ],
        compiler_params=pltpu.CompilerParams(
            collective_id=0, vmem_limit_bytes=100 * 1024 * 1024),
    )(x[0], Wq, kb, vb, Wo)
    return out[None]
